# baseline (speedup 1.0000x reference)
import functools

import jax
import jax.numpy as jnp
from jax import lax
from jax.experimental import pallas as pl
from jax.experimental.pallas import tpu as pltpu
from jax.experimental.pallas import tpu_sc as plsc

_LEVELS = 1000
_D = 1024
_L = 20
_B = 4096

_NC = 2
_NS = 16
_NW = _NC * _NS
_ROWS_PER_TILE = 32
_SPT = _B // _NW
_SUB = 16
_NSUB = _SPT // _SUB

_MAGIC = 2.0**23
_EXP1 = 0x3F800000
_SIGN = 0x80000000


def _lane_rotate(v, t):
    perm = (lax.iota(jnp.int32, 16) + (16 - t)) & 15
    dnums = lax.GatherDimensionNumbers(
        offset_dims=(),
        collapsed_slice_dims=(0,),
        start_index_map=(0,),
    )
    return lax.gather(
        v,
        perm[:, None],
        dnums,
        (1,),
        indices_are_sorted=False,
        unique_indices=True,
        mode=lax.GatherScatterMode.PROMISE_IN_BOUNDS,
    )


def _rot64(A, B, r):
    if r == 0:
        return A, B
    rr = jnp.uint32(r)
    rl = jnp.uint32(32 - r)
    return (A << rr) | (B >> rl), (B << rr) | (A >> rl)


def _pack_body(tbl_hbm, tp_hbm, tblv, varbuf, sem):
    wid = lax.axis_index("s") * _NC + lax.axis_index("c")
    r0 = jnp.minimum(wid * _ROWS_PER_TILE, _LEVELS - _ROWS_PER_TILE)
    pltpu.sync_copy(tbl_hbm.at[pl.ds(r0, _ROWS_PER_TILE)], tblv)

    zero16f = jnp.zeros((16,), jnp.float32)
    lane = lax.iota(jnp.int32, 16)

    def row_body(ll, carry):
        A = jnp.zeros((16,), jnp.uint32)
        B = jnp.zeros((16,), jnp.uint32)
        for p in range(32):
            bitsA = jnp.where(
                tblv[ll, pl.ds(16 * p, 16)] < zero16f,
                jnp.uint32(1 << p),
                jnp.uint32(0),
            )
            bitsB = jnp.where(
                tblv[ll, pl.ds(16 * (p + 32), 16)] < zero16f,
                jnp.uint32(1 << p),
                jnp.uint32(0),
            )
            A = A | bitsA
            B = B | bitsB
        for i in range(_L):
            s = _L - 1 - i
            q, t = divmod(s, 16)
            if t == 0:
                As, Bs = _rot64(A, B, q)
            else:
                Ag = _lane_rotate(A, t)
                Bg = _lane_rotate(B, t)
                Alo, Blo = _rot64(Ag, Bg, q)
                Ahi, Bhi = _rot64(Ag, Bg, q + 1)
                m = lane < t
                As = jnp.where(m, Ahi, Alo)
                Bs = jnp.where(m, Bhi, Blo)
            pr = ll * _L + i
            varbuf[pr, pl.ds(0, 16)] = As
            varbuf[pr, pl.ds(16, 16)] = Bs
        return carry

    lax.fori_loop(0, _ROWS_PER_TILE, row_body, 0)

    pltpu.async_copy(
        varbuf,
        tp_hbm.at[pl.ds(r0 * _L, _ROWS_PER_TILE * _L)],
        sem,
    ).wait()


def _encode_body(xf_hbm, tp_hbm, out_hbm, xv, cidx, rows_v, outbuf, gsem, os0, os1):
    wid = lax.axis_index("s") * _NC + lax.axis_index("c")
    b0 = wid * _SPT
    pltpu.sync_copy(xf_hbm.at[pl.ds(wid * (_SPT * _L), _SPT * _L)], xv)

    lane = lax.iota(jnp.int32, 16)
    nchunks = (_SPT * _L) // 16
    for c in range(nchunks):
        y = xv[pl.ds(16 * c, 16)] * jnp.float32(_LEVELS - 1)
        yr = (y + jnp.float32(_MAGIC)) - jnp.float32(_MAGIC)
        ri = yr.astype(jnp.int32)
        ri = jnp.minimum(jnp.maximum(ri, 0), _LEVELS - 1)
        pos = (lane + (16 * c)) % _L
        ci = ri * jnp.int32(_L) + pos
        cidx[c // 8, pl.ds((c % 8) * 16, 16)] = ci

    gathers = [
        pltpu.async_copy(tp_hbm.at[cidx.at[j]], rows_v.at[j], gsem)
        for j in range(_L)
    ]
    for g in gathers:
        g.wait()

    osems = [os0, os1]
    pending = [None, None]
    for sc in range(_NSUB):
        buf = sc % 2
        if pending[buf] is not None:
            pending[buf].wait()
            pending[buf] = None

        def sample_body(bsub, carry, _buf=buf, _sc=sc):
            f0 = (_sc * _SUB + bsub) * _L
            A = jnp.zeros((16,), jnp.uint32)
            B = jnp.zeros((16,), jnp.uint32)
            for t in range(_L):
                f = f0 + t
                j = lax.shift_right_logical(f, 7)
                r = f & 127
                A = A ^ rows_v[j, r, pl.ds(0, 16)]
                B = B ^ rows_v[j, r, pl.ds(16, 16)]
            for p in range(32):
                vA = lax.bitcast_convert_type(
                    ((A << jnp.uint32(31 - p)) & jnp.uint32(_SIGN))
                    | jnp.uint32(_EXP1),
                    jnp.float32,
                )
                vB = lax.bitcast_convert_type(
                    ((B << jnp.uint32(31 - p)) & jnp.uint32(_SIGN))
                    | jnp.uint32(_EXP1),
                    jnp.float32,
                )
                outbuf[_buf, bsub, pl.ds(16 * p, 16)] = vA
                outbuf[_buf, bsub, pl.ds(16 * (p + 32), 16)] = vB
            return carry

        lax.fori_loop(0, _SUB, sample_body, 0)
        pending[buf] = pltpu.async_copy(
            outbuf.at[buf],
            out_hbm.at[pl.ds(b0 + sc * _SUB, _SUB)],
            osems[buf],
        )
    for d in pending:
        if d is not None:
            d.wait()


@jax.jit
def kernel(x, table):
    mesh = plsc.VectorSubcoreMesh(core_axis_name="c", subcore_axis_name="s")

    pack = functools.partial(
        pl.kernel,
        mesh=mesh,
        out_type=jax.ShapeDtypeStruct((_L * _LEVELS, 32), jnp.uint32),
        scratch_types=[
            pltpu.VMEM((_ROWS_PER_TILE, _D), jnp.float32),
            pltpu.VMEM((_ROWS_PER_TILE * _L, 32), jnp.uint32),
            pltpu.SemaphoreType.DMA,
        ],
        compiler_params=pltpu.CompilerParams(use_tc_tiling_on_sc=False),
    )(_pack_body)
    tp = pack(table)

    encode = functools.partial(
        pl.kernel,
        mesh=mesh,
        out_type=jax.ShapeDtypeStruct((_B, _D), jnp.float32),
        scratch_types=[
            pltpu.VMEM((_SPT * _L,), jnp.float32),
            pltpu.VMEM((_L, _SPT), jnp.int32),
            pltpu.VMEM((_L, _SPT, 32), jnp.uint32),
            pltpu.VMEM((2, _SUB, _D), jnp.float32),
            pltpu.SemaphoreType.DMA,
            pltpu.SemaphoreType.DMA,
            pltpu.SemaphoreType.DMA,
        ],
        compiler_params=pltpu.CompilerParams(use_tc_tiling_on_sc=False),
    )(_encode_body)
    return encode(x.reshape(-1), tp)


# speedup vs baseline: 1.1484x; 1.1484x over previous
import functools

import jax
import jax.numpy as jnp
from jax import lax
from jax.experimental import pallas as pl
from jax.experimental.pallas import tpu as pltpu
from jax.experimental.pallas import tpu_sc as plsc

_LEVELS = 1000
_D = 1024
_L = 20
_B = 4096
_W = 128

_NC = 2
_NS = 16
_NW = _NC * _NS
_ROWS_PER_TILE = 32
_SPT = _B // _NW
_SUB = 16
_NSUB = _SPT // _SUB
_GB = 4
_NBATCH = _SPT // _GB
_RING = 8

_MAGIC = 2.0**23
_EXP1 = 0x3F800000
_SIGN = 0x80000000


def _lane_rotate(v, t):
    perm = (lax.iota(jnp.int32, 16) + (16 - t)) & 15
    dnums = lax.GatherDimensionNumbers(
        offset_dims=(),
        collapsed_slice_dims=(0,),
        start_index_map=(0,),
    )
    return lax.gather(
        v,
        perm[:, None],
        dnums,
        (1,),
        indices_are_sorted=False,
        unique_indices=True,
        mode=lax.GatherScatterMode.PROMISE_IN_BOUNDS,
    )


def _rot64(A, B, r):
    if r == 0:
        return A, B
    rr = jnp.uint32(r)
    rl = jnp.uint32(32 - r)
    return (A << rr) | (B >> rl), (B << rr) | (A >> rl)


def _pack_body(tbl_hbm, tp_hbm, tblv, varbuf, sem):
    wid = lax.axis_index("s") * _NC + lax.axis_index("c")
    r0 = jnp.minimum(wid * _ROWS_PER_TILE, _LEVELS - _ROWS_PER_TILE)
    pltpu.sync_copy(tbl_hbm.at[pl.ds(r0, _ROWS_PER_TILE)], tblv)

    zero16f = jnp.zeros((16,), jnp.float32)
    lane = lax.iota(jnp.int32, 16)

    def row_body(ll, carry):
        A = jnp.zeros((16,), jnp.uint32)
        B = jnp.zeros((16,), jnp.uint32)
        for p in range(32):
            bitsA = jnp.where(
                tblv[ll, pl.ds(16 * p, 16)] < zero16f,
                jnp.uint32(1 << p),
                jnp.uint32(0),
            )
            bitsB = jnp.where(
                tblv[ll, pl.ds(16 * (p + 32), 16)] < zero16f,
                jnp.uint32(1 << p),
                jnp.uint32(0),
            )
            A = A | bitsA
            B = B | bitsB
        for i in range(_L):
            s = _L - 1 - i
            q, t = divmod(s, 16)
            if t == 0:
                As, Bs = _rot64(A, B, q)
            else:
                Ag = _lane_rotate(A, t)
                Bg = _lane_rotate(B, t)
                Alo, Blo = _rot64(Ag, Bg, q)
                Ahi, Bhi = _rot64(Ag, Bg, q + 1)
                m = lane < t
                As = jnp.where(m, Ahi, Alo)
                Bs = jnp.where(m, Bhi, Blo)
            pr = ll * _L + i
            varbuf[pr, pl.ds(0, 16)] = As
            varbuf[pr, pl.ds(16, 16)] = Bs
        return carry

    lax.fori_loop(0, _ROWS_PER_TILE, row_body, 0)

    pltpu.async_copy(
        varbuf,
        tp_hbm.at[pl.ds(r0 * _L, _ROWS_PER_TILE * _L)],
        sem,
    ).wait()


def _encode_body(xf_hbm, tp_hbm, out_hbm, xv, cidx, rows_v, outbuf, gsem, os0, os1):
    wid = lax.axis_index("s") * _NC + lax.axis_index("c")
    b0 = wid * _SPT
    pltpu.sync_copy(xf_hbm.at[pl.ds(wid * (_SPT * _L), _SPT * _L)], xv)

    lane = lax.iota(jnp.int32, 16)
    nchunks = (_SPT * _L) // 16
    per_row = _GB * _L
    cpb = per_row // 16
    for c in range(nchunks):
        y = xv[pl.ds(16 * c, 16)] * jnp.float32(_LEVELS - 1)
        yr = (y + jnp.float32(_MAGIC)) - jnp.float32(_MAGIC)
        ri = yr.astype(jnp.int32)
        ri = jnp.minimum(jnp.maximum(ri, 0), _LEVELS - 1)
        pos = (lane + (16 * c)) % _L
        ci = ri * jnp.int32(_L) + pos
        cidx[c // cpb, pl.ds((c % cpb) * 16, 16)] = ci

    def issue(g):
        return pltpu.async_copy(
            tp_hbm.at[cidx.at[g]], rows_v.at[g % _RING], gsem
        )

    gdescs = {}
    for g in range(_RING):
        gdescs[g] = issue(g)

    osems = [os0, os1]
    pending = [None, None]
    for sc in range(_NSUB):
        buf = sc % 2
        for g in range(sc * (_SUB // _GB), (sc + 1) * (_SUB // _GB)):
            gdescs.pop(g).wait()
        if pending[buf] is not None:
            pending[buf].wait()
            pending[buf] = None

        def sample_body(bsub, carry, _buf=buf, _sc=sc):
            gslot = ((_sc * _SUB + bsub) // _GB) % _RING
            lrow = (bsub & (_GB - 1)) * _L
            loads = []
            for t in range(_L):
                loads.append(rows_v[gslot, lrow + t, pl.ds(0, 16)])
                loads.append(rows_v[gslot, lrow + t, pl.ds(16, 16)])
            avals = loads[0::2]
            bvals = loads[1::2]
            while len(avals) > 1:
                avals = [
                    avals[k] ^ avals[k + 1] if k + 1 < len(avals) else avals[k]
                    for k in range(0, len(avals), 2)
                ]
                bvals = [
                    bvals[k] ^ bvals[k + 1] if k + 1 < len(bvals) else bvals[k]
                    for k in range(0, len(bvals), 2)
                ]
            A, B = avals[0], bvals[0]
            for p in range(32):
                vA = lax.bitcast_convert_type(
                    ((A << jnp.uint32(31 - p)) & jnp.uint32(_SIGN))
                    | jnp.uint32(_EXP1),
                    jnp.float32,
                )
                vB = lax.bitcast_convert_type(
                    ((B << jnp.uint32(31 - p)) & jnp.uint32(_SIGN))
                    | jnp.uint32(_EXP1),
                    jnp.float32,
                )
                outbuf[_buf, bsub, pl.ds(16 * p, 16)] = vA
                outbuf[_buf, bsub, pl.ds(16 * (p + 32), 16)] = vB
            return carry

        lax.fori_loop(0, _SUB, sample_body, 0)
        for g in range((sc + 2) * (_SUB // _GB), (sc + 3) * (_SUB // _GB)):
            if g < _NBATCH:
                gdescs[g] = issue(g)
        pending[buf] = pltpu.async_copy(
            outbuf.at[buf],
            out_hbm.at[pl.ds(b0 + sc * _SUB, _SUB)],
            osems[buf],
        )
    for d in pending:
        if d is not None:
            d.wait()


@jax.jit
def kernel(x, table):
    mesh = plsc.VectorSubcoreMesh(core_axis_name="c", subcore_axis_name="s")

    pack = functools.partial(
        pl.kernel,
        mesh=mesh,
        out_type=jax.ShapeDtypeStruct((_L * _LEVELS, _W), jnp.uint32),
        scratch_types=[
            pltpu.VMEM((_ROWS_PER_TILE, _D), jnp.float32),
            pltpu.VMEM((_ROWS_PER_TILE * _L, _W), jnp.uint32),
            pltpu.SemaphoreType.DMA,
        ],
    )(_pack_body)
    tp = pack(table)

    encode = functools.partial(
        pl.kernel,
        mesh=mesh,
        out_type=jax.ShapeDtypeStruct((_B, _D), jnp.float32),
        scratch_types=[
            pltpu.VMEM((_SPT * _L,), jnp.float32),
            pltpu.VMEM((_NBATCH, _GB * _L), jnp.int32),
            pltpu.VMEM((_RING, _GB * _L, _W), jnp.uint32),
            pltpu.VMEM((2, _SUB, _D), jnp.float32),
            pltpu.SemaphoreType.DMA,
            pltpu.SemaphoreType.DMA,
            pltpu.SemaphoreType.DMA,
        ],
    )(_encode_body)
    return encode(x.reshape(-1), tp)
